# Initial kernel scaffold; baseline (speedup 1.0000x reference)
import jax
import jax.numpy as jnp
from jax.experimental import pallas as pl


def kernel(x, codes, params):
    raise NotImplementedError("write your pallas kernel here")


# speedup vs baseline: 9.5792x; 9.5792x over previous
import functools

import jax
import jax.numpy as jnp
import numpy as np
from jax.experimental import pallas as pl

GRID_SIZE = 48
RES = 0.25
SPACING = 2.0
HIDDEN = 128
N_LAYERS = 3
K = 16
N_CH = 8
BLK = 256


def _anchor_grid():
    half = (GRID_SIZE - 1) * RES / 2.0
    n = int(np.floor(2.0 * half / SPACING)) + 1
    lin = (np.arange(n) - (n - 1) / 2.0) * SPACING
    g = np.stack(np.meshgrid(lin, lin, lin, indexing='ij'), axis=-1).reshape(-1, 3)
    return np.asarray(g, dtype=np.float32)


def _htab_kernel(codes_ref, w_code_ref, b_code_ref, wm_n_ref, out_ref):
    h_a = codes_ref[0] @ w_code_ref[...]
    h_a = h_a + b_code_ref[...]
    h_a = h_a * jax.nn.sigmoid(h_a)
    for l in range(N_LAYERS):
        out_ref[0, :, l * HIDDEN:(l + 1) * HIDDEN] = jnp.dot(
            h_a, wm_n_ref[l], preferred_element_type=jnp.float32)


def _main_kernel(x_ref, htab_ref, anch_ref, anch_t_ref,
                 w_q_ref, b_q_ref, wm_q_ref, w_d2_ref, b_msg_ref, wx_ref,
                 wu1_ref, wu2_ref, b_upd_ref, w_out_ref, b_out_ref, out_ref):
    xr = x_ref[0]
    htab = htab_ref[0]
    blk = xr.shape[0]
    a_num = htab.shape[0]

    r0 = xr[:, 0:1] - anch_t_ref[0:1, :]
    r1 = xr[:, 1:2] - anch_t_ref[1:2, :]
    r2 = xr[:, 2:3] - anch_t_ref[2:3, :]
    d2a = r0 * r0 + r1 * r1 + r2 * r2

    iota = jax.lax.broadcasted_iota(jnp.int32, (blk, a_num), 1)
    hn = []
    pos = []
    for _ in range(K):
        v = jnp.min(d2a, axis=1, keepdims=True)
        am = jnp.min(jnp.where(d2a == v, iota, a_num), axis=1, keepdims=True)
        sel = iota == am
        mask = sel.astype(jnp.float32)
        hn.append(jnp.dot(mask, htab, preferred_element_type=jnp.float32))
        pos.append(jnp.dot(mask, anch_ref[...], preferred_element_type=jnp.float32))
        d2a = jnp.where(sel, 1e30, d2a)

    h_q = xr @ w_q_ref[...] + b_q_ref[...]
    h_q = h_q * jax.nn.sigmoid(h_q)
    x_cur = xr
    inv_k = jnp.float32(1.0 / K)

    for l in range(N_LAYERS):
        u = jnp.dot(h_q, wm_q_ref[l], preferred_element_type=jnp.float32)
        u = u + b_msg_ref[l]
        agg = jnp.zeros((blk, HIDDEN), jnp.float32)
        xacc = jnp.zeros((blk, 3), jnp.float32)
        for k in range(K):
            rel = x_cur - pos[k]
            d2 = jnp.sum(rel * rel, axis=1, keepdims=True)
            pre = u + hn[k][:, l * HIDDEN:(l + 1) * HIDDEN] + d2 * w_d2_ref[l]
            m = pre * jax.nn.sigmoid(pre)
            agg = agg + m
            cw = jnp.sum(m * wx_ref[l], axis=1, keepdims=True)
            xacc = xacc + rel * cw
        x_cur = x_cur + xacc * inv_k
        agg = agg * inv_k
        hq_new = (jnp.dot(h_q, wu1_ref[l], preferred_element_type=jnp.float32)
                  + jnp.dot(agg, wu2_ref[l], preferred_element_type=jnp.float32)
                  + b_upd_ref[l])
        h_q = hq_new * jax.nn.sigmoid(hq_new)

    out_ref[0] = jnp.dot(h_q, w_out_ref[...], preferred_element_type=jnp.float32) + b_out_ref[...]


@jax.jit
def kernel(x, codes, params):
    B, P, _ = x.shape
    anchors = _anchor_grid()
    A = anchors.shape[0]
    H = HIDDEN

    w_msg = params['W_msg']
    wm_q = w_msg[:, :H, :]
    wm_n = w_msg[:, H:2 * H, :]
    w_d2 = w_msg[:, 2 * H:2 * H + 1, :]
    b_msg = params['b_msg'][:, None, :]
    wu1 = params['W_upd'][:, :H, :]
    wu2 = params['W_upd'][:, H:, :]
    b_upd = params['b_upd'][:, None, :]
    wx = jnp.transpose(params['W_x'], (0, 2, 1))
    b_q = params['b_q'][None, :]
    b_code = params['b_code'][None, :]
    b_out = params['b_out'][None, :]
    anch = jnp.asarray(anchors)
    anch_t = jnp.asarray(anchors.T)

    htab = pl.pallas_call(
        _htab_kernel,
        grid=(B,),
        in_specs=[
            pl.BlockSpec((1, A, H), lambda b: (b, 0, 0)),
            pl.BlockSpec((H, H), lambda b: (0, 0)),
            pl.BlockSpec((1, H), lambda b: (0, 0)),
            pl.BlockSpec((N_LAYERS, H, H), lambda b: (0, 0, 0)),
        ],
        out_specs=pl.BlockSpec((1, A, N_LAYERS * H), lambda b: (b, 0, 0)),
        out_shape=jax.ShapeDtypeStruct((B, A, N_LAYERS * H), jnp.float32),
    )(codes, params['W_code'], b_code, wm_n)

    grid = (B, P // BLK)
    bcast2 = lambda b, i: (0, 0)
    bcast3 = lambda b, i: (0, 0, 0)
    out = pl.pallas_call(
        _main_kernel,
        grid=grid,
        in_specs=[
            pl.BlockSpec((1, BLK, 3), lambda b, i: (b, i, 0)),
            pl.BlockSpec((1, A, N_LAYERS * H), lambda b, i: (b, 0, 0)),
            pl.BlockSpec((A, 3), bcast2),
            pl.BlockSpec((3, A), bcast2),
            pl.BlockSpec((3, H), bcast2),
            pl.BlockSpec((1, H), bcast2),
            pl.BlockSpec((N_LAYERS, H, H), bcast3),
            pl.BlockSpec((N_LAYERS, 1, H), bcast3),
            pl.BlockSpec((N_LAYERS, 1, H), bcast3),
            pl.BlockSpec((N_LAYERS, 1, H), bcast3),
            pl.BlockSpec((N_LAYERS, H, H), bcast3),
            pl.BlockSpec((N_LAYERS, H, H), bcast3),
            pl.BlockSpec((N_LAYERS, 1, H), bcast3),
            pl.BlockSpec((H, N_CH), bcast2),
            pl.BlockSpec((1, N_CH), bcast2),
        ],
        out_specs=pl.BlockSpec((1, BLK, N_CH), lambda b, i: (b, i, 0)),
        out_shape=jax.ShapeDtypeStruct((B, P, N_CH), jnp.float32),
    )(x, htab, anch, anch_t,
      params['W_q'], b_q, wm_q, w_d2, b_msg, wx,
      wu1, wu2, b_upd, params['W_out'], b_out)
    return out


# speedup vs baseline: 10.9690x; 1.1451x over previous
import functools

import jax
import jax.numpy as jnp
import numpy as np
from jax.experimental import pallas as pl

GRID_SIZE = 48
RES = 0.25
SPACING = 2.0
HIDDEN = 128
N_LAYERS = 3
K = 16
N_CH = 8
BLK = 512


def _anchor_grid():
    half = (GRID_SIZE - 1) * RES / 2.0
    n = int(np.floor(2.0 * half / SPACING)) + 1
    lin = (np.arange(n) - (n - 1) / 2.0) * SPACING
    g = np.stack(np.meshgrid(lin, lin, lin, indexing='ij'), axis=-1).reshape(-1, 3)
    return np.asarray(g, dtype=np.float32)


def _htab_kernel(codes_ref, w_code_ref, b_code_ref, wm_n_ref, out_ref):
    h_a = codes_ref[0] @ w_code_ref[...]
    h_a = h_a + b_code_ref[...]
    h_a = h_a * jax.nn.sigmoid(h_a)
    for l in range(N_LAYERS):
        out_ref[0, :, l * HIDDEN:(l + 1) * HIDDEN] = jnp.dot(
            h_a, wm_n_ref[l], preferred_element_type=jnp.float32)


def _main_kernel(x_ref, htab_ref, anch_ref, anch_t_ref,
                 w_q_ref, b_q_ref, wm_q_ref, w_d2_ref, b_msg_ref, wx_ref,
                 wu1_ref, wu2_ref, b_upd_ref, w_out_ref, b_out_ref, out_ref):
    xr = x_ref[0]
    htab = htab_ref[0]
    blk = xr.shape[0]
    a_num = htab.shape[0]

    r0 = xr[:, 0:1] - anch_t_ref[0:1, :]
    r1 = xr[:, 1:2] - anch_t_ref[1:2, :]
    r2 = xr[:, 2:3] - anch_t_ref[2:3, :]
    d2a = r0 * r0 + r1 * r1 + r2 * r2

    iota = jax.lax.broadcasted_iota(jnp.int32, (blk, a_num), 1)
    masks = []
    for _ in range(K):
        v = jnp.min(d2a, axis=1, keepdims=True)
        am = jnp.min(jnp.where(d2a == v, iota, a_num), axis=1, keepdims=True)
        sel = iota == am
        masks.append(sel.astype(jnp.float32))
        d2a = jnp.where(sel, 1e30, d2a)
    mask_all = jnp.concatenate(masks, axis=0)
    hn_all = jnp.dot(mask_all, htab, preferred_element_type=jnp.float32)
    pos_all = jnp.dot(mask_all, anch_ref[...], preferred_element_type=jnp.float32)
    hn = [hn_all[k * blk:(k + 1) * blk] for k in range(K)]
    pos = [pos_all[k * blk:(k + 1) * blk] for k in range(K)]

    h_q = xr @ w_q_ref[...] + b_q_ref[...]
    h_q = h_q * jax.nn.sigmoid(h_q)
    x_cur = xr
    inv_k = jnp.float32(1.0 / K)

    for l in range(N_LAYERS):
        u = jnp.dot(h_q, wm_q_ref[l], preferred_element_type=jnp.float32)
        u = u + b_msg_ref[l]
        agg = jnp.zeros((blk, HIDDEN), jnp.float32)
        xacc = jnp.zeros((blk, 3), jnp.float32)
        for k in range(K):
            rel = x_cur - pos[k]
            d2 = jnp.sum(rel * rel, axis=1, keepdims=True)
            pre = u + hn[k][:, l * HIDDEN:(l + 1) * HIDDEN] + d2 * w_d2_ref[l]
            m = pre * jax.nn.sigmoid(pre)
            agg = agg + m
            cw = jnp.sum(m * wx_ref[l], axis=1, keepdims=True)
            xacc = xacc + rel * cw
        x_cur = x_cur + xacc * inv_k
        agg = agg * inv_k
        hq_new = (jnp.dot(h_q, wu1_ref[l], preferred_element_type=jnp.float32)
                  + jnp.dot(agg, wu2_ref[l], preferred_element_type=jnp.float32)
                  + b_upd_ref[l])
        h_q = hq_new * jax.nn.sigmoid(hq_new)

    out_ref[0] = jnp.dot(h_q, w_out_ref[...], preferred_element_type=jnp.float32) + b_out_ref[...]


@jax.jit
def kernel(x, codes, params):
    B, P, _ = x.shape
    anchors = _anchor_grid()
    A = anchors.shape[0]
    H = HIDDEN

    w_msg = params['W_msg']
    wm_q = w_msg[:, :H, :]
    wm_n = w_msg[:, H:2 * H, :]
    w_d2 = w_msg[:, 2 * H:2 * H + 1, :]
    b_msg = params['b_msg'][:, None, :]
    wu1 = params['W_upd'][:, :H, :]
    wu2 = params['W_upd'][:, H:, :]
    b_upd = params['b_upd'][:, None, :]
    wx = jnp.transpose(params['W_x'], (0, 2, 1))
    b_q = params['b_q'][None, :]
    b_code = params['b_code'][None, :]
    b_out = params['b_out'][None, :]
    anch = jnp.asarray(anchors)
    anch_t = jnp.asarray(anchors.T)

    htab = pl.pallas_call(
        _htab_kernel,
        grid=(B,),
        in_specs=[
            pl.BlockSpec((1, A, H), lambda b: (b, 0, 0)),
            pl.BlockSpec((H, H), lambda b: (0, 0)),
            pl.BlockSpec((1, H), lambda b: (0, 0)),
            pl.BlockSpec((N_LAYERS, H, H), lambda b: (0, 0, 0)),
        ],
        out_specs=pl.BlockSpec((1, A, N_LAYERS * H), lambda b: (b, 0, 0)),
        out_shape=jax.ShapeDtypeStruct((B, A, N_LAYERS * H), jnp.float32),
    )(codes, params['W_code'], b_code, wm_n)

    grid = (B, P // BLK)
    bcast2 = lambda b, i: (0, 0)
    bcast3 = lambda b, i: (0, 0, 0)
    out = pl.pallas_call(
        _main_kernel,
        grid=grid,
        in_specs=[
            pl.BlockSpec((1, BLK, 3), lambda b, i: (b, i, 0)),
            pl.BlockSpec((1, A, N_LAYERS * H), lambda b, i: (b, 0, 0)),
            pl.BlockSpec((A, 3), bcast2),
            pl.BlockSpec((3, A), bcast2),
            pl.BlockSpec((3, H), bcast2),
            pl.BlockSpec((1, H), bcast2),
            pl.BlockSpec((N_LAYERS, H, H), bcast3),
            pl.BlockSpec((N_LAYERS, 1, H), bcast3),
            pl.BlockSpec((N_LAYERS, 1, H), bcast3),
            pl.BlockSpec((N_LAYERS, 1, H), bcast3),
            pl.BlockSpec((N_LAYERS, H, H), bcast3),
            pl.BlockSpec((N_LAYERS, H, H), bcast3),
            pl.BlockSpec((N_LAYERS, 1, H), bcast3),
            pl.BlockSpec((H, N_CH), bcast2),
            pl.BlockSpec((1, N_CH), bcast2),
        ],
        out_specs=pl.BlockSpec((1, BLK, N_CH), lambda b, i: (b, i, 0)),
        out_shape=jax.ShapeDtypeStruct((B, P, N_CH), jnp.float32),
    )(x, htab, anch, anch_t,
      params['W_q'], b_q, wm_q, w_d2, b_msg, wx,
      wu1, wu2, b_upd, params['W_out'], b_out)
    return out


# speedup vs baseline: 13.6162x; 1.2413x over previous
import functools

import jax
import jax.numpy as jnp
import numpy as np
from jax.experimental import pallas as pl

GRID_SIZE = 48
RES = 0.25
SPACING = 2.0
HIDDEN = 128
N_LAYERS = 3
K = 16
N_CH = 8
BLK = 512


def _anchor_grid():
    half = (GRID_SIZE - 1) * RES / 2.0
    n = int(np.floor(2.0 * half / SPACING)) + 1
    lin = (np.arange(n) - (n - 1) / 2.0) * SPACING
    g = np.stack(np.meshgrid(lin, lin, lin, indexing='ij'), axis=-1).reshape(-1, 3)
    return np.asarray(g, dtype=np.float32)


def _htab_kernel(codes_ref, w_code_ref, b_code_ref, wm_n_ref, out_ref):
    h_a = codes_ref[0] @ w_code_ref[...]
    h_a = h_a + b_code_ref[...]
    h_a = h_a * jax.nn.sigmoid(h_a)
    for l in range(N_LAYERS):
        out_ref[0, :, l * HIDDEN:(l + 1) * HIDDEN] = jnp.dot(
            h_a, wm_n_ref[l], preferred_element_type=jnp.float32)


def _main_kernel(x_ref, htab_ref, anch_ref, anch_t_ref,
                 w_q_ref, b_q_ref, wm_q_ref, w_d2_ref, b_msg_ref, wx_ref,
                 wu1_ref, wu2_ref, b_upd_ref, w_out_ref, b_out_ref, out_ref):
    xr = x_ref[0]
    htab = htab_ref[0]
    blk = xr.shape[0]
    a_num = htab.shape[0]

    r0 = xr[:, 0:1] - anch_t_ref[0:1, :]
    r1 = xr[:, 1:2] - anch_t_ref[1:2, :]
    r2 = xr[:, 2:3] - anch_t_ref[2:3, :]
    d2a = r0 * r0 + r1 * r1 + r2 * r2

    masks = []
    for _ in range(K):
        v = jnp.min(d2a, axis=1, keepdims=True)
        sel = d2a == v
        masks.append(sel.astype(jnp.float32))
        d2a = jnp.where(sel, 1e30, d2a)
    mask_all = jnp.concatenate(masks, axis=0)
    hn_all = jnp.dot(mask_all, htab, preferred_element_type=jnp.float32)
    pos_all = jnp.dot(mask_all, anch_ref[...], preferred_element_type=jnp.float32)
    hn = [hn_all[k * blk:(k + 1) * blk] for k in range(K)]
    pos = [pos_all[k * blk:(k + 1) * blk] for k in range(K)]

    h_q = xr @ w_q_ref[...] + b_q_ref[...]
    h_q = h_q * jax.nn.sigmoid(h_q)
    x_cur = xr
    inv_k = jnp.float32(1.0 / K)

    for l in range(N_LAYERS):
        u = jnp.dot(h_q, wm_q_ref[l], preferred_element_type=jnp.float32)
        u = u + b_msg_ref[l]
        agg = jnp.zeros((blk, HIDDEN), jnp.float32)
        xacc = jnp.zeros((blk, 3), jnp.float32)
        for k in range(K):
            rel = x_cur - pos[k]
            d2 = jnp.sum(rel * rel, axis=1, keepdims=True)
            pre = u + hn[k][:, l * HIDDEN:(l + 1) * HIDDEN] + d2 * w_d2_ref[l]
            m = pre * jax.nn.sigmoid(pre)
            agg = agg + m
            cw = jnp.sum(m * wx_ref[l], axis=1, keepdims=True)
            xacc = xacc + rel * cw
        x_cur = x_cur + xacc * inv_k
        agg = agg * inv_k
        hq_new = (jnp.dot(h_q, wu1_ref[l], preferred_element_type=jnp.float32)
                  + jnp.dot(agg, wu2_ref[l], preferred_element_type=jnp.float32)
                  + b_upd_ref[l])
        h_q = hq_new * jax.nn.sigmoid(hq_new)

    out_ref[0] = jnp.dot(h_q, w_out_ref[...], preferred_element_type=jnp.float32) + b_out_ref[...]


@jax.jit
def kernel(x, codes, params):
    B, P, _ = x.shape
    anchors = _anchor_grid()
    A = anchors.shape[0]
    H = HIDDEN

    w_msg = params['W_msg']
    wm_q = w_msg[:, :H, :]
    wm_n = w_msg[:, H:2 * H, :]
    w_d2 = w_msg[:, 2 * H:2 * H + 1, :]
    b_msg = params['b_msg'][:, None, :]
    wu1 = params['W_upd'][:, :H, :]
    wu2 = params['W_upd'][:, H:, :]
    b_upd = params['b_upd'][:, None, :]
    wx = jnp.transpose(params['W_x'], (0, 2, 1))
    b_q = params['b_q'][None, :]
    b_code = params['b_code'][None, :]
    b_out = params['b_out'][None, :]
    anch = jnp.asarray(anchors)
    anch_t = jnp.asarray(anchors.T)

    htab = pl.pallas_call(
        _htab_kernel,
        grid=(B,),
        in_specs=[
            pl.BlockSpec((1, A, H), lambda b: (b, 0, 0)),
            pl.BlockSpec((H, H), lambda b: (0, 0)),
            pl.BlockSpec((1, H), lambda b: (0, 0)),
            pl.BlockSpec((N_LAYERS, H, H), lambda b: (0, 0, 0)),
        ],
        out_specs=pl.BlockSpec((1, A, N_LAYERS * H), lambda b: (b, 0, 0)),
        out_shape=jax.ShapeDtypeStruct((B, A, N_LAYERS * H), jnp.float32),
    )(codes, params['W_code'], b_code, wm_n)

    grid = (B, P // BLK)
    bcast2 = lambda b, i: (0, 0)
    bcast3 = lambda b, i: (0, 0, 0)
    out = pl.pallas_call(
        _main_kernel,
        grid=grid,
        in_specs=[
            pl.BlockSpec((1, BLK, 3), lambda b, i: (b, i, 0)),
            pl.BlockSpec((1, A, N_LAYERS * H), lambda b, i: (b, 0, 0)),
            pl.BlockSpec((A, 3), bcast2),
            pl.BlockSpec((3, A), bcast2),
            pl.BlockSpec((3, H), bcast2),
            pl.BlockSpec((1, H), bcast2),
            pl.BlockSpec((N_LAYERS, H, H), bcast3),
            pl.BlockSpec((N_LAYERS, 1, H), bcast3),
            pl.BlockSpec((N_LAYERS, 1, H), bcast3),
            pl.BlockSpec((N_LAYERS, 1, H), bcast3),
            pl.BlockSpec((N_LAYERS, H, H), bcast3),
            pl.BlockSpec((N_LAYERS, H, H), bcast3),
            pl.BlockSpec((N_LAYERS, 1, H), bcast3),
            pl.BlockSpec((H, N_CH), bcast2),
            pl.BlockSpec((1, N_CH), bcast2),
        ],
        out_specs=pl.BlockSpec((1, BLK, N_CH), lambda b, i: (b, i, 0)),
        out_shape=jax.ShapeDtypeStruct((B, P, N_CH), jnp.float32),
    )(x, htab, anch, anch_t,
      params['W_q'], b_q, wm_q, w_d2, b_msg, wx,
      wu1, wu2, b_upd, params['W_out'], b_out)
    return out
